# baseline (speedup 1.0000x reference)
import functools

import jax
import jax.numpy as jnp
from jax import lax
from jax.experimental import pallas as pl
from jax.experimental.pallas import tpu as pltpu
from jax.experimental.pallas import tpu_sc as plsc

EPS = 1e-9
LANES = 16
NC, NS = 2, 16
NW = NC * NS
CH = 80
GRP = CH // LANES


def _quad16(srows_v, drows_v, tab_v, si, di, e0, lanes):
    rowi = e0 + lanes
    pa = jnp.zeros((LANES,), jnp.float32)
    pb = jnp.zeros((LANES,), jnp.float32)
    feat = lanes
    for _ in range(64):
        sword = plsc.load_gather(srows_v, [rowi, feat])
        tword = plsc.load_gather(drows_v, [rowi, feat])
        prod = plsc.bitcast(sword, jnp.bfloat16) * plsc.bitcast(tword, jnp.bfloat16)
        u0, u1 = plsc.unpack(prod, format=plsc.PackFormat.INTERLEAVED)
        pa = pa + u0
        pb = pb + u1
        feat = (feat + 1) & 63
    p = pa + pb
    si2 = si * 2
    di2 = di * 2
    ns = plsc.load_gather(tab_v, [si2])
    cs = plsc.load_gather(tab_v, [si2 + 1])
    nd = plsc.load_gather(tab_v, [di2])
    cd = plsc.load_gather(tab_v, [di2 + 1])
    dp = p - 2.0 * cs * cd
    denom = jnp.maximum(jnp.abs(ns * nd), EPS)
    return 1.0 - dp * dp / denom


def _sc_body(z_hbm, tab_hbm, edge_hbm, bs_hbm, iota_hbm,
             pos_out, cnt_out, neg_out,
             sidx_v, didx_v, tab_v, z_sh,
             srows0_v, drows0_v, srows1_v, drows1_v,
             bs_v, iota_v, accp_v, accc_v, accn_v, sem0, sem1,
             *, per_stream, pos_count, neg_per_tile, n_nodes):
    c = lax.axis_index("c")
    s = lax.axis_index("s")
    wid = s * NC + c

    @pl.when(s == 0)
    def _():
        pltpu.sync_copy(z_hbm, z_sh)
    n_chunks = per_stream // CH
    n_stream = per_stream * NW

    pltpu.sync_copy(edge_hbm.at[pl.ds(wid * per_stream, per_stream)], sidx_v)
    pltpu.sync_copy(edge_hbm.at[pl.ds(n_stream + wid * per_stream, per_stream)],
                    didx_v)
    pltpu.sync_copy(bs_hbm, bs_v)
    pltpu.sync_copy(iota_hbm, iota_v)
    pltpu.sync_copy(tab_hbm, tab_v)
    bs_vec = bs_v[...]
    lanes_rt = iota_v[...]
    zero = jnp.zeros((LANES,), jnp.float32)
    accp_v[...] = zero
    accc_v[...] = zero
    accn_v[...] = zero
    plsc.subcore_barrier()

    def issue(k, srows, drows, sem):
        si = sidx_v.at[pl.ds(k * CH, CH)]
        di = didx_v.at[pl.ds(k * CH, CH)]
        pltpu.async_copy(z_sh.at[si], srows, sem)
        pltpu.async_copy(z_sh.at[di], drows, sem)

    def drain(srows, drows, sem):
        pltpu.make_async_copy(z_hbm.at[pl.ds(0, CH)], srows, sem).wait()
        pltpu.make_async_copy(z_hbm.at[pl.ds(0, CH)], drows, sem).wait()

    def compute(k, srows, drows):
        def grp(g, _):
            e0 = g * LANES
            si = sidx_v[pl.ds(k * CH + e0, LANES)]
            di = didx_v[pl.ds(k * CH + e0, LANES)]
            q = _quad16(srows, drows, tab_v, si, di, e0, lanes_rt)
            pid = k * CH + e0 + lanes_rt
            is_pos = pid < pos_count
            inb = (si < bs_vec) & (di < bs_vec)
            mfp = jnp.where(is_pos & inb, 1.0, 0.0)
            gid = wid * neg_per_tile + pid - pos_count
            in_neg = (~is_pos) & (pid < pos_count + neg_per_tile)
            mfn = jnp.where(in_neg & (gid < n_nodes), 1.0, 0.0)
            accp_v[...] = accp_v[...] + jnp.minimum(q, 10.0) * mfp
            accc_v[...] = accc_v[...] + mfp
            accn_v[...] = accn_v[...] + jnp.maximum(1.0 - q, 0.0) * mfn
            return 0

        lax.fori_loop(0, GRP, grp, 0)

    issue(0, srows0_v, drows0_v, sem0)

    def pair(j, _):
        k0 = 2 * j
        issue(k0 + 1, srows1_v, drows1_v, sem1)
        drain(srows0_v, drows0_v, sem0)
        compute(k0, srows0_v, drows0_v)
        issue(k0 + 2, srows0_v, drows0_v, sem0)
        drain(srows1_v, drows1_v, sem1)
        compute(k0 + 1, srows1_v, drows1_v)
        return 0

    lax.fori_loop(0, (n_chunks - 1) // 2, pair, 0)
    drain(srows0_v, drows0_v, sem0)
    compute(n_chunks - 1, srows0_v, drows0_v)

    pltpu.sync_copy(accp_v, pos_out.at[wid])
    pltpu.sync_copy(accc_v, cnt_out.at[wid])
    pltpu.sync_copy(accn_v, neg_out.at[wid])


@functools.partial(jax.jit, static_argnames=("pos_count", "neg_per_tile"))
def _uhg_loss_sc(z, tab, edge_stream, bs_vec, *, pos_count, neg_per_tile):
    n_nodes, d_model = z.shape
    per_stream = edge_stream.shape[0] // 2 // NW

    body = functools.partial(
        _sc_body, per_stream=per_stream, pos_count=pos_count,
        neg_per_tile=neg_per_tile, n_nodes=n_nodes)
    out_sds = jax.ShapeDtypeStruct((NW, LANES), jnp.float32)
    mesh = plsc.VectorSubcoreMesh(core_axis_name="c", subcore_axis_name="s")
    rows_t = pltpu.VMEM((CH, d_model), jnp.int32)
    f = pl.kernel(
        body,
        out_type=(out_sds, out_sds, out_sds),
        mesh=mesh,
        compiler_params=pltpu.CompilerParams(
            needs_layout_passes=False, use_tc_tiling_on_sc=False),
        scratch_types=[
            pltpu.VMEM((per_stream,), jnp.int32),
            pltpu.VMEM((per_stream,), jnp.int32),
            pltpu.VMEM((n_nodes * 2,), jnp.float32),
            pltpu.VMEM_SHARED((n_nodes, d_model), jnp.int32),
            rows_t, rows_t, rows_t, rows_t,
            pltpu.VMEM((LANES,), jnp.int32),
            pltpu.VMEM((LANES,), jnp.int32),
            pltpu.VMEM((LANES,), jnp.float32),
            pltpu.VMEM((LANES,), jnp.float32),
            pltpu.VMEM((LANES,), jnp.float32),
            pltpu.SemaphoreType.DMA,
            pltpu.SemaphoreType.DMA,
        ],
    )
    return f(z, tab, edge_stream, bs_vec,
             jnp.arange(LANES, dtype=jnp.int32))


def kernel(z, edge_index, batch_size):
    n_nodes = z.shape[0]
    n_edges = edge_index.shape[1]
    pos_count = n_edges // NW
    neg = jax.random.randint(jax.random.key(42), (2, n_nodes), 0, batch_size,
                             dtype=jnp.int32)
    neg_per_tile = -(-n_nodes // NW)
    neg_padded = jnp.pad(neg, ((0, 0), (0, NW * neg_per_tile - n_nodes)))
    bs_vec = jnp.full((LANES,), batch_size, dtype=jnp.int32)
    nt = jnp.sum(z[:, :-1] ** 2, axis=1) - z[:, -1] ** 2
    tab = jnp.stack([nt, z[:, -1]], axis=1).reshape(-1)

    raw = pos_count + neg_per_tile
    pad = -raw % CH
    stream = jnp.concatenate([
        edge_index.reshape(2, NW, pos_count),
        neg_padded.reshape(2, NW, neg_per_tile),
        jnp.zeros((2, NW, pad), jnp.int32),
    ], axis=2).reshape(-1)

    zw = jax.lax.bitcast_convert_type(
        z.astype(jnp.bfloat16).reshape(n_nodes, 64, 2), jnp.int32)
    pos_s, cnt_s, neg_s = _uhg_loss_sc(
        zw, tab, stream, bs_vec,
        pos_count=pos_count, neg_per_tile=neg_per_tile)

    pos_sum = jnp.sum(pos_s)
    count = jnp.sum(cnt_s)
    neg_sum = jnp.sum(neg_s)
    pos_loss = pos_sum / count
    neg_loss = neg_sum / n_nodes
    total = 0.5 * (pos_loss + neg_loss) + 0.01 * pos_loss
    return jnp.clip(total, 0.0, 1000.0)


# speedup vs baseline: 1.0224x; 1.0224x over previous
import functools

import jax
import jax.numpy as jnp
from jax import lax
from jax.experimental import pallas as pl
from jax.experimental.pallas import tpu as pltpu
from jax.experimental.pallas import tpu_sc as plsc

EPS = 1e-9
LANES = 16
NC, NS = 2, 16
NW = NC * NS
CH = 80
GRP = CH // LANES


def _quad16(srows_v, drows_v, tab_v, si, di, e0, lanes):
    rowi = e0 + lanes
    pa = jnp.zeros((LANES,), jnp.float32)
    pb = jnp.zeros((LANES,), jnp.float32)
    feat = lanes
    for _ in range(64):
        sword = plsc.load_gather(srows_v, [rowi, feat])
        tword = plsc.load_gather(drows_v, [rowi, feat])
        prod = plsc.bitcast(sword, jnp.bfloat16) * plsc.bitcast(tword, jnp.bfloat16)
        u0, u1 = plsc.unpack(prod, format=plsc.PackFormat.INTERLEAVED)
        pa = pa + u0
        pb = pb + u1
        feat = (feat + 1) & 63
    p = pa + pb
    si2 = si * 2
    di2 = di * 2
    ns = plsc.load_gather(tab_v, [si2])
    cs = plsc.load_gather(tab_v, [si2 + 1])
    nd = plsc.load_gather(tab_v, [di2])
    cd = plsc.load_gather(tab_v, [di2 + 1])
    dp = p - 2.0 * cs * cd
    denom = jnp.maximum(jnp.abs(ns * nd), EPS)
    return 1.0 - dp * dp / denom


def _sc_body(z_hbm, tab_hbm, pos_hbm, neg_hbm, bs_hbm, iota_hbm,
             pos_out, cnt_out, neg_out,
             sidx_v, didx_v, tab_v, z_sh,
             srows0_v, drows0_v, srows1_v, drows1_v,
             bs_v, iota_v, accp_v, accc_v, accn_v, sem0, sem1,
             *, per_tile, neg_per_tile, n_nodes):
    c = lax.axis_index("c")
    s = lax.axis_index("s")
    wid = s * NC + c

    @pl.when(s == 0)
    def _():
        pltpu.sync_copy(z_hbm, z_sh)
    n_pos_chunks = per_tile // CH
    n_chunks = n_pos_chunks + neg_per_tile // CH
    n_edges = per_tile * NW
    n_neg = neg_per_tile * NW

    pltpu.sync_copy(pos_hbm.at[pl.ds(wid * per_tile, per_tile)],
                    sidx_v.at[pl.ds(0, per_tile)])
    pltpu.sync_copy(pos_hbm.at[pl.ds(n_edges + wid * per_tile, per_tile)],
                    didx_v.at[pl.ds(0, per_tile)])
    pltpu.sync_copy(neg_hbm.at[pl.ds(wid * neg_per_tile, neg_per_tile)],
                    sidx_v.at[pl.ds(per_tile, neg_per_tile)])
    pltpu.sync_copy(neg_hbm.at[pl.ds(n_neg + wid * neg_per_tile, neg_per_tile)],
                    didx_v.at[pl.ds(per_tile, neg_per_tile)])
    pltpu.sync_copy(bs_hbm, bs_v)
    pltpu.sync_copy(iota_hbm, iota_v)
    pltpu.sync_copy(tab_hbm, tab_v)
    bs_vec = bs_v[...]
    lanes_rt = iota_v[...]
    zero = jnp.zeros((LANES,), jnp.float32)
    accp_v[...] = zero
    accc_v[...] = zero
    accn_v[...] = zero
    plsc.subcore_barrier()

    def issue(k, srows, drows, sem):
        si = sidx_v.at[pl.ds(k * CH, CH)]
        di = didx_v.at[pl.ds(k * CH, CH)]
        pltpu.async_copy(z_sh.at[si], srows, sem)
        pltpu.async_copy(z_sh.at[di], drows, sem)

    def drain(srows, drows, sem):
        pltpu.make_async_copy(z_hbm.at[pl.ds(0, CH)], srows, sem).wait()
        pltpu.make_async_copy(z_hbm.at[pl.ds(0, CH)], drows, sem).wait()

    def compute(k, srows, drows):
        is_pos = k < n_pos_chunks
        neg_off = wid * neg_per_tile + k * CH - per_tile

        def grp(g, _):
            e0 = g * LANES
            si = sidx_v[pl.ds(k * CH + e0, LANES)]
            di = didx_v[pl.ds(k * CH + e0, LANES)]
            q = _quad16(srows, drows, tab_v, si, di, e0, lanes_rt)
            inb = (si < bs_vec) & (di < bs_vec)
            pos_vec = jnp.full((LANES,), is_pos)
            mfp = jnp.where(pos_vec & inb, 1.0, 0.0)
            gid = neg_off + e0 + lanes_rt
            mfn = jnp.where((~pos_vec) & (gid < n_nodes), 1.0, 0.0)
            accp_v[...] = accp_v[...] + jnp.minimum(q, 10.0) * mfp
            accc_v[...] = accc_v[...] + mfp
            accn_v[...] = accn_v[...] + jnp.maximum(1.0 - q, 0.0) * mfn
            return 0

        lax.fori_loop(0, GRP, grp, 0)

    issue(0, srows0_v, drows0_v, sem0)

    def pair(j, _):
        k0 = 2 * j
        issue(k0 + 1, srows1_v, drows1_v, sem1)
        drain(srows0_v, drows0_v, sem0)
        compute(k0, srows0_v, drows0_v)
        issue(k0 + 2, srows0_v, drows0_v, sem0)
        drain(srows1_v, drows1_v, sem1)
        compute(k0 + 1, srows1_v, drows1_v)
        return 0

    lax.fori_loop(0, (n_chunks - 1) // 2, pair, 0)
    drain(srows0_v, drows0_v, sem0)
    compute(n_chunks - 1, srows0_v, drows0_v)

    pltpu.sync_copy(accp_v, pos_out.at[wid])
    pltpu.sync_copy(accc_v, cnt_out.at[wid])
    pltpu.sync_copy(accn_v, neg_out.at[wid])


@jax.jit
def _uhg_loss_sc(z, tab, edge_index, neg_padded, bs_vec):
    n_nodes, d_model = z.shape
    n_edges = edge_index.shape[0] // 2
    per_tile = n_edges // NW
    neg_per_tile = neg_padded.shape[0] // 2 // NW

    body = functools.partial(
        _sc_body, per_tile=per_tile, neg_per_tile=neg_per_tile,
        n_nodes=n_nodes)
    out_sds = jax.ShapeDtypeStruct((NW, LANES), jnp.float32)
    mesh = plsc.VectorSubcoreMesh(core_axis_name="c", subcore_axis_name="s")
    rows_t = pltpu.VMEM((CH, d_model), jnp.int32)
    f = pl.kernel(
        body,
        out_type=(out_sds, out_sds, out_sds),
        mesh=mesh,
        compiler_params=pltpu.CompilerParams(
            needs_layout_passes=False, use_tc_tiling_on_sc=False),
        scratch_types=[
            pltpu.VMEM((per_tile + neg_per_tile,), jnp.int32),
            pltpu.VMEM((per_tile + neg_per_tile,), jnp.int32),
            pltpu.VMEM((n_nodes * 2,), jnp.float32),
            pltpu.VMEM_SHARED((n_nodes, d_model), jnp.int32),
            rows_t, rows_t, rows_t, rows_t,
            pltpu.VMEM((LANES,), jnp.int32),
            pltpu.VMEM((LANES,), jnp.int32),
            pltpu.VMEM((LANES,), jnp.float32),
            pltpu.VMEM((LANES,), jnp.float32),
            pltpu.VMEM((LANES,), jnp.float32),
            pltpu.SemaphoreType.DMA,
            pltpu.SemaphoreType.DMA,
        ],
    )
    return f(z, tab, edge_index, neg_padded, bs_vec,
             jnp.arange(LANES, dtype=jnp.int32))


def kernel(z, edge_index, batch_size):
    n_nodes = z.shape[0]
    neg = jax.random.randint(jax.random.key(42), (2, n_nodes), 0, batch_size,
                             dtype=jnp.int32)
    neg_cap = ((n_nodes + NW * CH - 1) // (NW * CH)) * (NW * CH)
    neg_padded = jnp.pad(neg, ((0, 0), (0, neg_cap - n_nodes)))
    bs_vec = jnp.full((LANES,), batch_size, dtype=jnp.int32)
    nt = jnp.sum(z[:, :-1] ** 2, axis=1) - z[:, -1] ** 2
    tab = jnp.stack([nt, z[:, -1]], axis=1).reshape(-1)

    zw = jax.lax.bitcast_convert_type(
        z.astype(jnp.bfloat16).reshape(n_nodes, 64, 2), jnp.int32)
    pos_s, cnt_s, neg_s = _uhg_loss_sc(
        zw, tab, edge_index.reshape(-1), neg_padded.reshape(-1), bs_vec)

    pos_sum = jnp.sum(pos_s)
    count = jnp.sum(cnt_s)
    neg_sum = jnp.sum(neg_s)
    pos_loss = pos_sum / count
    neg_loss = neg_sum / n_nodes
    total = 0.5 * (pos_loss + neg_loss) + 0.01 * pos_loss
    return jnp.clip(total, 0.0, 1000.0)
